# baseline (speedup 1.0000x reference)
import jax
import jax.numpy as jnp
from jax.experimental import pallas as pl
from jax.experimental.pallas import tpu as pltpu

_ROWS = 160000
_COLS = 256
_BLOCK = 14544


def _copy_body(x_ref, o_ref):
    o_ref[...] = x_ref[...]


def kernel(x_j):
    grid = (pl.cdiv(_ROWS, _BLOCK),)
    return pl.pallas_call(
        _copy_body,
        grid=grid,
        in_specs=[pl.BlockSpec((_BLOCK, _COLS), lambda i: (i, 0))],
        out_specs=pl.BlockSpec((_BLOCK, _COLS), lambda i: (i, 0)),
        out_shape=jax.ShapeDtypeStruct(x_j.shape, x_j.dtype),
        compiler_params=pltpu.CompilerParams(
            dimension_semantics=("parallel",),
        ),
    )(x_j)


# speedup vs baseline: 1.0099x; 1.0099x over previous
import jax
import jax.numpy as jnp
from jax.experimental import pallas as pl
from jax.experimental.pallas import tpu as pltpu

_ROWS = 160000
_COLS = 256
_BLOCK = 14208


def _copy_body(x_ref, o_ref):
    o_ref[...] = x_ref[...]


def kernel(x_j):
    grid = (pl.cdiv(_ROWS, _BLOCK),)
    return pl.pallas_call(
        _copy_body,
        grid=grid,
        in_specs=[pl.BlockSpec((_BLOCK, _COLS), lambda i: (i, 0))],
        out_specs=pl.BlockSpec((_BLOCK, _COLS), lambda i: (i, 0)),
        out_shape=jax.ShapeDtypeStruct(x_j.shape, x_j.dtype),
        compiler_params=pltpu.CompilerParams(
            dimension_semantics=("parallel",),
        ),
    )(x_j)


# speedup vs baseline: 1.0107x; 1.0008x over previous
import jax
import jax.numpy as jnp
from jax.experimental import pallas as pl
from jax.experimental.pallas import tpu as pltpu

_ROWS = 160000
_COLS = 256
_BLOCK = 14080


def _copy_body(x_ref, o_ref):
    o_ref[...] = x_ref[...]


def kernel(x_j):
    grid = (pl.cdiv(_ROWS, _BLOCK),)
    return pl.pallas_call(
        _copy_body,
        grid=grid,
        in_specs=[pl.BlockSpec((_BLOCK, _COLS), lambda i: (i, 0))],
        out_specs=pl.BlockSpec((_BLOCK, _COLS), lambda i: (i, 0)),
        out_shape=jax.ShapeDtypeStruct(x_j.shape, x_j.dtype),
        compiler_params=pltpu.CompilerParams(
            dimension_semantics=("parallel",),
        ),
    )(x_j)


# speedup vs baseline: 1.0115x; 1.0009x over previous
import jax
import jax.numpy as jnp
from jax.experimental import pallas as pl
from jax.experimental.pallas import tpu as pltpu

_ROWS = 160000
_COLS = 256
_BLOCK = 14000


def _copy_body(x_ref, o_ref):
    o_ref[...] = x_ref[...]


def kernel(x_j):
    grid = (pl.cdiv(_ROWS, _BLOCK),)
    return pl.pallas_call(
        _copy_body,
        grid=grid,
        in_specs=[pl.BlockSpec((_BLOCK, _COLS), lambda i: (i, 0))],
        out_specs=pl.BlockSpec((_BLOCK, _COLS), lambda i: (i, 0)),
        out_shape=jax.ShapeDtypeStruct(x_j.shape, x_j.dtype),
        compiler_params=pltpu.CompilerParams(
            dimension_semantics=("parallel",),
        ),
    )(x_j)
